# baseline (speedup 1.0000x reference)
import jax
import jax.numpy as jnp
from jax import lax
from jax.experimental import pallas as pl
from jax.experimental.pallas import tpu as pltpu
from jax.experimental.pallas import tpu_sc as plsc

N = 1048576
NBINS = 10
TAIL = 5.0
MINW = 0.001
MINH = 0.001
MIND = 0.001
NWORKERS = 32
CHUNK = 2048
NGROUPS = CHUNK // 16

_WSCALE = (1.0 - MINW * NBINS) * (2.0 * TAIL)
_HSCALE = (1.0 - MINH * NBINS) * (2.0 * TAIL)


def _tree_reduce(vals, op):
    vals = list(vals)
    while len(vals) > 1:
        nxt = [op(vals[i], vals[i + 1]) for i in range(0, len(vals) - 1, 2)]
        if len(vals) % 2:
            nxt.append(vals[-1])
        vals = nxt
    return vals[0]


def _log1p_small(v):
    s = v / (2.0 + v)
    s2 = s * s
    return 2.0 * s * (1.0 + s2 * (1.0 / 3.0 + s2 * (1.0 / 5.0 + s2 * (1.0 / 7.0))))


def _softplus(u):
    t = jnp.exp(-jnp.abs(u))
    return jnp.maximum(u, 0.0) + _log1p_small(t)


def _sqrt_nn(v):
    i = lax.bitcast_convert_type(v, jnp.int32)
    i = 0x5F3759DF - lax.shift_right_logical(i, 1)
    r = lax.bitcast_convert_type(i, jnp.float32)
    r = r * (1.5 - 0.5 * v * r * r)
    r = r * (1.5 - 0.5 * v * r * r)
    r = r * (1.5 - 0.5 * v * r * r)
    return v * r


def _spline_group(x, uwk, uhk, udb, lane_e):
    f32 = jnp.float32

    ew = [jnp.exp(v) for v in uwk]
    eh = [jnp.exp(v) for v in uhk]
    sw = _tree_reduce(ew, jnp.add)
    sh = _tree_reduce(eh, jnp.add)
    rw = _WSCALE / sw
    rh = _HSCALE / sh
    cws = [ew[0]]
    chs = [eh[0]]
    for k in range(1, NBINS):
        cws.append(cws[k - 1] + ew[k])
        chs.append(chs[k - 1] + eh[k])

    one_i = jnp.full((16,), 1, jnp.int32)
    zero_i = jnp.full((16,), 0, jnp.int32)
    step = MINH * 2.0 * TAIL
    idx = zero_i
    for k in range(1, NBINS):
        knot = (step * k - TAIL) + rh * chs[k - 1]
        idx = idx + jnp.where(x >= knot, one_i, zero_i)

    mks = [idx == k for k in range(NBINS)]
    zero_f = jnp.full((16,), 0.0, f32)
    pw0 = zero_f
    ph0 = zero_f
    for k in range(1, NBINS):
        pw0 = jnp.where(mks[k], cws[k - 1], pw0)
        ph0 = jnp.where(mks[k], chs[k - 1], ph0)
    pw1 = cws[NBINS - 1]
    ph1 = chs[NBINS - 1]
    for k in range(NBINS - 1):
        pw1 = jnp.where(mks[k], cws[k], pw1)
        ph1 = jnp.where(mks[k], chs[k], ph1)
    idx_f = idx.astype(f32)
    ch_lo = (step * idx_f - TAIL) + rh * ph0
    cw_lo = (step * idx_f - TAIL) + rw * pw0
    heights = step + rh * (ph1 - ph0)
    widths = step + rw * (pw1 - pw0)

    d_lo_raw = plsc.load_gather(udb, [idx, lane_e])
    d_hi_raw = plsc.load_gather(udb, [idx + 1, lane_e])
    one_f = jnp.full((16,), 1.0, f32)
    d_lo = jnp.where(mks[0], one_f, MIND + _softplus(d_lo_raw))
    d_hi = jnp.where(mks[NBINS - 1], one_f, MIND + _softplus(d_hi_raw))

    delta = heights / widths
    dx = x - ch_lo
    two = d_lo + d_hi - 2.0 * delta
    aq = dx * two + heights * (delta - d_lo)
    bq = heights * d_lo - dx * two
    cq = -delta * dx
    disc = bq * bq - 4.0 * aq * cq
    root = (2.0 * cq) / (-bq - _sqrt_nn(jnp.maximum(disc, 0.0)))
    return root * widths + cw_lo


def _make_run(total):
    welems = total // NWORKERS
    nchunks = welems // CHUNK

    def tec_kernel(x_hbm, uw_hbm, uh_hbm, ud_hbm, out_hbm, xb, uwb, uhb, udb, ob, sem):
        c = lax.axis_index("c")
        s = lax.axis_index("s")
        wid = s * 2 + c
        wbase = wid * welems
        lane = lax.iota(jnp.int32, 16)

        def chunk_body(ci, carry):
            ebase = wbase + ci * CHUNK
            copies = [pltpu.make_async_copy(x_hbm.at[pl.ds(ebase, CHUNK)], xb, sem)]
            for k in range(NBINS):
                copies.append(pltpu.make_async_copy(
                    uw_hbm.at[pl.ds(k * total + ebase, CHUNK)], uwb.at[k], sem))
                copies.append(pltpu.make_async_copy(
                    uh_hbm.at[pl.ds(k * total + ebase, CHUNK)], uhb.at[k], sem))
            for k in range(NBINS + 1):
                copies.append(pltpu.make_async_copy(
                    ud_hbm.at[pl.ds(k * total + ebase, CHUNK)], udb.at[k], sem))
            for cp in copies:
                cp.start()
            for cp in copies:
                cp.wait()

            def group_body(g, carry2):
                b = g * 16
                lane_e = b + lane
                uwk = [uwb[k, pl.ds(b, 16)] for k in range(NBINS)]
                uhk = [uhb[k, pl.ds(b, 16)] for k in range(NBINS)]
                x = xb[pl.ds(b, 16)]
                out = _spline_group(x, uwk, uhk, udb, lane_e)
                ob[pl.ds(b, 16)] = out
                return carry2

            lax.fori_loop(0, NGROUPS, group_body, 0)
            pltpu.sync_copy(ob, out_hbm.at[pl.ds(ebase, CHUNK)])
            return carry

        lax.fori_loop(0, nchunks, chunk_body, 0)

    mesh = plsc.VectorSubcoreMesh(core_axis_name="c", subcore_axis_name="s")
    return pl.kernel(
        tec_kernel,
        out_type=jax.ShapeDtypeStruct((total,), jnp.float32),
        mesh=mesh,
        compiler_params=pltpu.CompilerParams(
            needs_layout_passes=False, use_tc_tiling_on_sc=False),
        scratch_types=[
            pltpu.VMEM((CHUNK,), jnp.float32),
            pltpu.VMEM((NBINS, CHUNK), jnp.float32),
            pltpu.VMEM((NBINS, CHUNK), jnp.float32),
            pltpu.VMEM((NBINS + 1, CHUNK), jnp.float32),
            pltpu.VMEM((CHUNK,), jnp.float32),
            pltpu.SemaphoreType.DMA,
        ],
    )


NSPLIT = 16
NPART = N // NSPLIT


@jax.jit
def _run(x, uw, uh, ud):
    run_part = _make_run(NPART)
    outs = []
    for h in range(NSPLIT):
        sl = slice(h * NPART, (h + 1) * NPART)
        uwt = jnp.concatenate([uw[0, 0, sl, k] for k in range(NBINS)])
        uht = jnp.concatenate([uh[0, 0, sl, k] for k in range(NBINS)])
        udt = jnp.concatenate([ud[0, 0, sl, k] for k in range(NBINS + 1)])
        outs.append(run_part(x[sl], uwt, uht, udt))
    return jnp.concatenate(outs)


def kernel(inputs, unnormalized_widths, unnormalized_heights, unnormalized_derivatives):
    x = inputs.reshape(N)
    out = _run(x, unnormalized_widths, unnormalized_heights,
               unnormalized_derivatives)
    return out.reshape(1, 1, N)


# speedup vs baseline: 1.0501x; 1.0501x over previous
import jax
import jax.numpy as jnp
from jax import lax
from jax.experimental import pallas as pl
from jax.experimental.pallas import tpu as pltpu
from jax.experimental.pallas import tpu_sc as plsc

N = 1048576
NBINS = 10
TAIL = 5.0
MINW = 0.001
MINH = 0.001
MIND = 0.001
NWORKERS = 32
CHUNK = 2048
NGROUPS = CHUNK // 16

_WSCALE = (1.0 - MINW * NBINS) * (2.0 * TAIL)
_HSCALE = (1.0 - MINH * NBINS) * (2.0 * TAIL)


def _tree_reduce(vals, op):
    vals = list(vals)
    while len(vals) > 1:
        nxt = [op(vals[i], vals[i + 1]) for i in range(0, len(vals) - 1, 2)]
        if len(vals) % 2:
            nxt.append(vals[-1])
        vals = nxt
    return vals[0]


def _log1p_small(v):
    s = v / (2.0 + v)
    s2 = s * s
    return 2.0 * s * (1.0 + s2 * (1.0 / 3.0 + s2 * (1.0 / 5.0 + s2 * (1.0 / 7.0))))


def _softplus(u):
    t = jnp.exp(-jnp.abs(u))
    return jnp.maximum(u, 0.0) + _log1p_small(t)


def _sqrt_nn(v):
    i = lax.bitcast_convert_type(v, jnp.int32)
    i = 0x5F3759DF - lax.shift_right_logical(i, 1)
    r = lax.bitcast_convert_type(i, jnp.float32)
    r = r * (1.5 - 0.5 * v * r * r)
    r = r * (1.5 - 0.5 * v * r * r)
    r = r * (1.5 - 0.5 * v * r * r)
    return v * r


def _spline_group(x, uwk, uhk, udb, lane_e):
    f32 = jnp.float32

    ew = [jnp.exp(v) for v in uwk]
    eh = [jnp.exp(v) for v in uhk]
    sw = _tree_reduce(ew, jnp.add)
    sh = _tree_reduce(eh, jnp.add)
    rw = _WSCALE / sw
    rh = _HSCALE / sh
    cws = [ew[0]]
    chs = [eh[0]]
    for k in range(1, NBINS):
        cws.append(cws[k - 1] + ew[k])
        chs.append(chs[k - 1] + eh[k])

    one_i = jnp.full((16,), 1, jnp.int32)
    zero_i = jnp.full((16,), 0, jnp.int32)
    step = MINH * 2.0 * TAIL
    idx = zero_i
    for k in range(1, NBINS):
        knot = (step * k - TAIL) + rh * chs[k - 1]
        idx = idx + jnp.where(x >= knot, one_i, zero_i)

    mks = [idx == k for k in range(NBINS)]
    zero_f = jnp.full((16,), 0.0, f32)
    pw0 = zero_f
    ph0 = zero_f
    for k in range(1, NBINS):
        pw0 = jnp.where(mks[k], cws[k - 1], pw0)
        ph0 = jnp.where(mks[k], chs[k - 1], ph0)
    pw1 = cws[NBINS - 1]
    ph1 = chs[NBINS - 1]
    for k in range(NBINS - 1):
        pw1 = jnp.where(mks[k], cws[k], pw1)
        ph1 = jnp.where(mks[k], chs[k], ph1)
    idx_f = idx.astype(f32)
    ch_lo = (step * idx_f - TAIL) + rh * ph0
    cw_lo = (step * idx_f - TAIL) + rw * pw0
    heights = step + rh * (ph1 - ph0)
    widths = step + rw * (pw1 - pw0)

    d_lo_raw = plsc.load_gather(udb, [idx, lane_e])
    d_hi_raw = plsc.load_gather(udb, [idx + 1, lane_e])
    one_f = jnp.full((16,), 1.0, f32)
    d_lo = jnp.where(mks[0], one_f, MIND + _softplus(d_lo_raw))
    d_hi = jnp.where(mks[NBINS - 1], one_f, MIND + _softplus(d_hi_raw))

    delta = heights / widths
    dx = x - ch_lo
    two = d_lo + d_hi - 2.0 * delta
    aq = dx * two + heights * (delta - d_lo)
    bq = heights * d_lo - dx * two
    cq = -delta * dx
    disc = bq * bq - 4.0 * aq * cq
    root = (2.0 * cq) / (-bq - _sqrt_nn(jnp.maximum(disc, 0.0)))
    return root * widths + cw_lo


def _make_run(total):
    welems = total // NWORKERS
    nchunks = welems // CHUNK

    def tec_kernel(x_hbm, uw_hbm, uh_hbm, ud_hbm, out_hbm, xb, uwb, uhb, udb, ob, sem):
        c = lax.axis_index("c")
        s = lax.axis_index("s")
        wid = s * 2 + c
        wbase = wid * welems
        lane = lax.iota(jnp.int32, 16)

        def chunk_body(ci, carry):
            ebase = wbase + ci * CHUNK
            copies = [pltpu.make_async_copy(x_hbm.at[pl.ds(ebase, CHUNK)], xb, sem)]
            for k in range(NBINS):
                copies.append(pltpu.make_async_copy(
                    uw_hbm.at[pl.ds(k * total + ebase, CHUNK)], uwb.at[k], sem))
                copies.append(pltpu.make_async_copy(
                    uh_hbm.at[pl.ds(k * total + ebase, CHUNK)], uhb.at[k], sem))
            for k in range(NBINS + 1):
                copies.append(pltpu.make_async_copy(
                    ud_hbm.at[pl.ds(k * total + ebase, CHUNK)], udb.at[k], sem))
            for cp in copies:
                cp.start()
            for cp in copies:
                cp.wait()

            def group_body(g, carry2):
                b = g * 16
                lane_e = b + lane
                uwk = [uwb[k, pl.ds(b, 16)] for k in range(NBINS)]
                uhk = [uhb[k, pl.ds(b, 16)] for k in range(NBINS)]
                x = xb[pl.ds(b, 16)]
                out = _spline_group(x, uwk, uhk, udb, lane_e)
                ob[pl.ds(b, 16)] = out
                return carry2

            lax.fori_loop(0, NGROUPS, group_body, 0)
            pltpu.sync_copy(ob, out_hbm.at[pl.ds(ebase, CHUNK)])
            return carry

        lax.fori_loop(0, nchunks, chunk_body, 0)

    mesh = plsc.VectorSubcoreMesh(core_axis_name="c", subcore_axis_name="s")
    return pl.kernel(
        tec_kernel,
        out_type=jax.ShapeDtypeStruct((total,), jnp.float32),
        mesh=mesh,
        compiler_params=pltpu.CompilerParams(
            needs_layout_passes=False, use_tc_tiling_on_sc=False),
        scratch_types=[
            pltpu.VMEM((CHUNK,), jnp.float32),
            pltpu.VMEM((NBINS, CHUNK), jnp.float32),
            pltpu.VMEM((NBINS, CHUNK), jnp.float32),
            pltpu.VMEM((NBINS + 1, CHUNK), jnp.float32),
            pltpu.VMEM((CHUNK,), jnp.float32),
            pltpu.SemaphoreType.DMA,
        ],
    )


NSPLIT = 8
NPART = N // NSPLIT


@jax.jit
def _run(x, uw, uh, ud):
    run_part = _make_run(NPART)
    outs = []
    for h in range(NSPLIT):
        sl = slice(h * NPART, (h + 1) * NPART)
        uwt = jnp.concatenate([uw[0, 0, sl, k] for k in range(NBINS)])
        uht = jnp.concatenate([uh[0, 0, sl, k] for k in range(NBINS)])
        udt = jnp.concatenate([ud[0, 0, sl, k] for k in range(NBINS + 1)])
        outs.append(run_part(x[sl], uwt, uht, udt))
    return jnp.concatenate(outs)


def kernel(inputs, unnormalized_widths, unnormalized_heights, unnormalized_derivatives):
    x = inputs.reshape(N)
    out = _run(x, unnormalized_widths, unnormalized_heights,
               unnormalized_derivatives)
    return out.reshape(1, 1, N)
